# baseline (device time: 50139 ns/iter reference)
import math

import jax
import jax.numpy as jnp
from jax import lax
from jax.experimental import pallas as pl
from jax.experimental.pallas import tpu as pltpu

N_DEV = 4
B_LOC = 2
SQ = 128
D = 512
H_LOC = 4
DH = 64
ROWS = B_LOC * SQ
HD_LOC = H_LOC * DH


def _rope_tables():
    pos = lax.broadcasted_iota(jnp.int32, (ROWS, HD_LOC), 0) % SQ
    c = lax.broadcasted_iota(jnp.int32, (ROWS, HD_LOC), 1) % DH
    k = c - (c % 2)
    inv = jnp.exp(k.astype(jnp.float32) * (-math.log(10000.0) / DH))
    ang = pos.astype(jnp.float32) * inv
    cos = jnp.cos(ang)
    sin = jnp.sin(ang)

    i = lax.broadcasted_iota(jnp.int32, (HD_LOC, HD_LOC), 0)
    j = lax.broadcasted_iota(jnp.int32, (HD_LOC, HD_LOC), 1)
    up = ((j == i + 1) & (i % 2 == 0)).astype(jnp.float32)
    dn = ((j == i - 1) & (i % 2 == 1)).astype(jnp.float32)
    rmat = (up - dn).astype(jnp.bfloat16)
    return cos, sin, rmat


def kernel(x, Wq, Wk, Wv, Wo):
    x2 = x.reshape(ROWS, D)

    def body(x_ref, wq_ref, wk_ref, wv_ref, wo_ref, out_ref,
             xg, acc, rs_stage, ctx,
             ag_send_sems, ag_recv_sems, rs_send_sems, rs_recv_sems):
        my = lax.axis_index("i")
        left = (my + N_DEV - 1) % N_DEV
        right = (my + 1) % N_DEV

        barrier = pltpu.get_barrier_semaphore()
        for nbr in (left, right):
            pl.semaphore_signal(barrier, inc=1, device_id=(nbr,),
                                device_id_type=pl.DeviceIdType.MESH)
        pl.semaphore_wait(barrier, 2)

        wq = wq_ref[...].astype(jnp.bfloat16)
        wk = wk_ref[...].astype(jnp.bfloat16)
        wv = wv_ref[...].astype(jnp.bfloat16)
        wo = wo_ref[...].astype(jnp.bfloat16)
        cos, sin, rmat = _rope_tables()

        xg[0, :, :] = x_ref[...].astype(jnp.bfloat16)

        for h in range(N_DEV - 1):
            rdma = pltpu.make_async_remote_copy(
                src_ref=xg.at[h],
                dst_ref=xg.at[h + 1],
                send_sem=ag_send_sems.at[h],
                recv_sem=ag_recv_sems.at[h + 1],
                device_id=(right,),
                device_id_type=pl.DeviceIdType.MESH,
            )
            rdma.start()
            rdma.wait()

        def partial_out(xr):
            q = jnp.dot(xr, wq, preferred_element_type=jnp.float32)
            k = jnp.dot(xr, wk, preferred_element_type=jnp.float32)
            v = jnp.dot(xr, wv,
                        preferred_element_type=jnp.float32).astype(jnp.bfloat16)

            def rope(t):
                tr = jnp.dot(t.astype(jnp.bfloat16), rmat,
                             preferred_element_type=jnp.float32)
                return (t * cos + tr * sin).astype(jnp.bfloat16)

            qr = rope(q)
            kr = rope(k)
            for b in range(B_LOC):
                for hh in range(H_LOC):
                    rsl = slice(b * SQ, (b + 1) * SQ)
                    csl = slice(hh * DH, (hh + 1) * DH)
                    s = lax.dot_general(
                        qr[rsl, csl], kr[rsl, csl],
                        (((1,), (1,)), ((), ())),
                        preferred_element_type=jnp.float32) * 0.125
                    m = jnp.max(s, axis=-1, keepdims=True)
                    w = jnp.exp(s - m)
                    w = w / jnp.sum(w, axis=-1, keepdims=True)
                    ctx[rsl, csl] = jnp.dot(
                        w.astype(jnp.bfloat16), v[rsl, csl],
                        preferred_element_type=jnp.float32).astype(jnp.bfloat16)
            return jnp.dot(ctx[...], wo,
                           preferred_element_type=jnp.float32).astype(jnp.bfloat16)

        for r in range(N_DEV):
            acc[r, :, :] = partial_out(xg[r])

        for s in range(N_DEV - 1):
            rdma = pltpu.make_async_remote_copy(
                src_ref=acc.at[s + 1],
                dst_ref=rs_stage.at[s],
                send_sem=rs_send_sems.at[s],
                recv_sem=rs_recv_sems.at[s],
                device_id=(right,),
                device_id_type=pl.DeviceIdType.MESH,
            )
            rdma.start()
            rdma.wait()
            dst = (s + 2) % N_DEV
            acc[dst, :, :] = acc[dst] + rs_stage[s]

        out_ref[...] = acc[0].astype(jnp.float32)

    out = pl.pallas_call(
        body,
        out_shape=jax.ShapeDtypeStruct((ROWS, D), jnp.float32),
        in_specs=[pl.BlockSpec(memory_space=pltpu.VMEM)] * 5,
        out_specs=pl.BlockSpec(memory_space=pltpu.VMEM),
        scratch_shapes=[
            pltpu.VMEM((N_DEV, ROWS, D), jnp.bfloat16),
            pltpu.VMEM((N_DEV, ROWS, D), jnp.bfloat16),
            pltpu.VMEM((N_DEV - 1, ROWS, D), jnp.bfloat16),
            pltpu.VMEM((ROWS, HD_LOC), jnp.bfloat16),
            pltpu.SemaphoreType.DMA((N_DEV - 1,)),
            pltpu.SemaphoreType.DMA((N_DEV,)),
            pltpu.SemaphoreType.DMA((N_DEV - 1,)),
            pltpu.SemaphoreType.DMA((N_DEV - 1,)),
        ],
        compiler_params=pltpu.CompilerParams(collective_id=0),
    )(x2, Wq, Wk, Wv, Wo)
    return out.reshape(B_LOC, SQ, D)


# device time: 23234 ns/iter; 2.1580x vs baseline; 2.1580x over previous
import math

import jax
import jax.numpy as jnp
from jax import lax
from jax.experimental import pallas as pl
from jax.experimental.pallas import tpu as pltpu

N_DEV = 4
B_LOC = 2
SQ = 128
D = 512
H_LOC = 4
DH = 64
ROWS = B_LOC * SQ
HD_LOC = H_LOC * DH

_REMOTE_ORDER = (1, 3, 2)


def _rope_tables():
    pos = lax.broadcasted_iota(jnp.int32, (ROWS, HD_LOC), 0) % SQ
    c = lax.broadcasted_iota(jnp.int32, (ROWS, HD_LOC), 1) % DH
    k = c - (c % 2)
    inv = jnp.exp(k.astype(jnp.float32) * (-math.log(10000.0) / DH))
    ang = pos.astype(jnp.float32) * inv
    cos = jnp.cos(ang)
    sin = jnp.sin(ang)

    i = lax.broadcasted_iota(jnp.int32, (HD_LOC, HD_LOC), 0)
    j = lax.broadcasted_iota(jnp.int32, (HD_LOC, HD_LOC), 1)
    up = ((j == i + 1) & (i % 2 == 0)).astype(jnp.float32)
    dn = ((j == i - 1) & (i % 2 == 1)).astype(jnp.float32)
    rmat = (up - dn).astype(jnp.bfloat16)
    return cos, sin, rmat


def kernel(x, Wq, Wk, Wv, Wo):
    x2 = x.reshape(ROWS, D)

    def body(x_ref, wq_ref, wk_ref, wv_ref, wo_ref, out_ref,
             xg, acc, rs_stage, ctx,
             ag_send_sems, ag_recv_sems, rs_send_sems, rs_recv_sems):
        my = lax.axis_index("i")

        barrier = pltpu.get_barrier_semaphore()
        for d in range(1, N_DEV):
            pl.semaphore_signal(barrier, inc=1,
                                device_id=((my + d) % N_DEV,),
                                device_id_type=pl.DeviceIdType.MESH)
        pl.semaphore_wait(barrier, N_DEV - 1)

        xg[0, :, :] = x_ref[...].astype(jnp.bfloat16)

        ag_sends = []
        for d in range(1, N_DEV):
            rdma = pltpu.make_async_remote_copy(
                src_ref=xg.at[0],
                dst_ref=xg.at[d],
                send_sem=ag_send_sems.at[d - 1],
                recv_sem=ag_recv_sems.at[d - 1],
                device_id=((my + d) % N_DEV,),
                device_id_type=pl.DeviceIdType.MESH,
            )
            rdma.start()
            ag_sends.append(rdma)

        wq = wq_ref[...].astype(jnp.bfloat16)
        wk = wk_ref[...].astype(jnp.bfloat16)
        wv = wv_ref[...].astype(jnp.bfloat16)
        wo = wo_ref[...].astype(jnp.bfloat16)
        cos, sin, rmat = _rope_tables()

        def partial_out(xr):
            q = jnp.dot(xr, wq, preferred_element_type=jnp.float32)
            k = jnp.dot(xr, wk, preferred_element_type=jnp.float32)
            v = jnp.dot(xr, wv,
                        preferred_element_type=jnp.float32).astype(jnp.bfloat16)

            def rope(t):
                tr = jnp.dot(t.astype(jnp.bfloat16), rmat,
                             preferred_element_type=jnp.float32)
                return (t * cos + tr * sin).astype(jnp.bfloat16)

            qr = rope(q)
            kr = rope(k)
            for b in range(B_LOC):
                for hh in range(H_LOC):
                    rsl = slice(b * SQ, (b + 1) * SQ)
                    csl = slice(hh * DH, (hh + 1) * DH)
                    s = lax.dot_general(
                        qr[rsl, csl], kr[rsl, csl],
                        (((1,), (1,)), ((), ())),
                        preferred_element_type=jnp.float32) * 0.125
                    m = jnp.max(s, axis=-1, keepdims=True)
                    w = jnp.exp(s - m)
                    w = w / jnp.sum(w, axis=-1, keepdims=True)
                    ctx[rsl, csl] = jnp.dot(
                        w.astype(jnp.bfloat16), v[rsl, csl],
                        preferred_element_type=jnp.float32).astype(jnp.bfloat16)
            return jnp.dot(ctx[...], wo,
                           preferred_element_type=jnp.float32).astype(jnp.bfloat16)

        acc[0, :, :] = partial_out(xg[0])

        rs_sends = []
        for r in _REMOTE_ORDER:
            recv = pltpu.make_async_remote_copy(
                src_ref=xg.at[r],
                dst_ref=xg.at[r],
                send_sem=ag_send_sems.at[r - 1],
                recv_sem=ag_recv_sems.at[r - 1],
                device_id=(my,),
                device_id_type=pl.DeviceIdType.MESH,
            )
            recv.wait_recv()
            acc[r, :, :] = partial_out(xg[r])
            rdma = pltpu.make_async_remote_copy(
                src_ref=acc.at[r],
                dst_ref=rs_stage.at[r - 1],
                send_sem=rs_send_sems.at[r - 1],
                recv_sem=rs_recv_sems.at[r - 1],
                device_id=((my + N_DEV - r) % N_DEV,),
                device_id_type=pl.DeviceIdType.MESH,
            )
            rdma.start()
            rs_sends.append(rdma)

        for r in _REMOTE_ORDER:
            recv = pltpu.make_async_remote_copy(
                src_ref=rs_stage.at[r - 1],
                dst_ref=rs_stage.at[r - 1],
                send_sem=rs_send_sems.at[r - 1],
                recv_sem=rs_recv_sems.at[r - 1],
                device_id=(my,),
                device_id_type=pl.DeviceIdType.MESH,
            )
            recv.wait_recv()

        out_ref[...] = (acc[0].astype(jnp.float32)
                        + rs_stage[0].astype(jnp.float32)
                        + rs_stage[1].astype(jnp.float32)
                        + rs_stage[2].astype(jnp.float32))

        for rdma in ag_sends + rs_sends:
            rdma.wait_send()

    out = pl.pallas_call(
        body,
        out_shape=jax.ShapeDtypeStruct((ROWS, D), jnp.float32),
        in_specs=[pl.BlockSpec(memory_space=pltpu.VMEM)] * 5,
        out_specs=pl.BlockSpec(memory_space=pltpu.VMEM),
        scratch_shapes=[
            pltpu.VMEM((N_DEV, ROWS, D), jnp.bfloat16),
            pltpu.VMEM((N_DEV, ROWS, D), jnp.bfloat16),
            pltpu.VMEM((N_DEV - 1, ROWS, D), jnp.bfloat16),
            pltpu.VMEM((ROWS, HD_LOC), jnp.bfloat16),
            pltpu.SemaphoreType.DMA((N_DEV - 1,)),
            pltpu.SemaphoreType.DMA((N_DEV - 1,)),
            pltpu.SemaphoreType.DMA((N_DEV - 1,)),
            pltpu.SemaphoreType.DMA((N_DEV - 1,)),
        ],
        compiler_params=pltpu.CompilerParams(collective_id=0),
    )(x2, Wq, Wk, Wv, Wo)
    return out.reshape(B_LOC, SQ, D)


# device time: 1947 ns/iter; 25.7519x vs baseline; 11.9332x over previous
import math

import numpy as np

import jax
import jax.numpy as jnp
from jax import lax
from jax.experimental import pallas as pl
from jax.experimental.pallas import tpu as pltpu

N_DEV = 4
B_LOC = 2
SQ = 128
D = 512
H_LOC = 4
DH = 64
ROWS = B_LOC * SQ
HD_LOC = H_LOC * DH
SCALE = 0.125

_REMOTE_ORDER = (1, 3, 2)


def _rope_tables():
    pos = np.arange(ROWS)[:, None] % SQ
    c = np.arange(HD_LOC)[None, :] % DH
    k = c - (c % 2)
    inv = np.exp(k * (-math.log(10000.0) / DH))
    ang = (pos * inv).astype(np.float32)
    cos = np.cos(ang)
    sin = np.sin(ang)

    i = np.arange(HD_LOC)[:, None]
    j = np.arange(HD_LOC)[None, :]
    up = ((j == i + 1) & (i % 2 == 0)).astype(np.float32)
    dn = ((j == i - 1) & (i % 2 == 1)).astype(np.float32)
    rmat = up - dn
    return cos, sin, rmat


_COS, _SIN, _RMAT = _rope_tables()


def kernel(x, Wq, Wk, Wv, Wo):
    x2 = x.reshape(ROWS, D)
    cos_q = jnp.asarray(_COS * SCALE)
    sin_q = jnp.asarray(_SIN * SCALE)
    cos_k = jnp.asarray(_COS)
    sin_k = jnp.asarray(_SIN)
    rmat = jnp.asarray(_RMAT.astype(np.float32)).astype(jnp.bfloat16)

    def body(x_ref, wq_ref, wk_ref, wv_ref, wo_ref,
             cos_q_ref, sin_q_ref, cos_k_ref, sin_k_ref, rmat_ref,
             out_ref,
             xg, acc, rs_stage, ctx,
             ag_send_sems, ag_recv_sems, rs_send_sems, rs_recv_sems):
        my = lax.axis_index("i")

        barrier = pltpu.get_barrier_semaphore()
        for d in range(1, N_DEV):
            pl.semaphore_signal(barrier, inc=1,
                                device_id=((my + d) % N_DEV,),
                                device_id_type=pl.DeviceIdType.MESH)
        pl.semaphore_wait(barrier, N_DEV - 1)

        xg[0, :, :] = x_ref[...].astype(jnp.bfloat16)

        ag_sends = []
        for d in range(1, N_DEV):
            rdma = pltpu.make_async_remote_copy(
                src_ref=xg.at[0],
                dst_ref=xg.at[d],
                send_sem=ag_send_sems.at[d - 1],
                recv_sem=ag_recv_sems.at[d - 1],
                device_id=((my + d) % N_DEV,),
                device_id_type=pl.DeviceIdType.MESH,
            )
            rdma.start()
            ag_sends.append(rdma)

        wq = wq_ref[...].astype(jnp.bfloat16)
        wk = wk_ref[...].astype(jnp.bfloat16)
        wv = wv_ref[...].astype(jnp.bfloat16)
        wo = wo_ref[...].astype(jnp.bfloat16)
        rm = rmat_ref[...]

        def partial_out(xr):
            q = jnp.dot(xr, wq, preferred_element_type=jnp.float32)
            k = jnp.dot(xr, wk, preferred_element_type=jnp.float32)
            v = jnp.dot(xr, wv,
                        preferred_element_type=jnp.float32).astype(jnp.bfloat16)

            def rope(t, cos_t, sin_t):
                tr = jnp.dot(t.astype(jnp.bfloat16), rm,
                             preferred_element_type=jnp.float32)
                return (t * cos_t + tr * sin_t).astype(jnp.bfloat16)

            qr = rope(q, cos_q_ref[...], sin_q_ref[...])
            kr = rope(k, cos_k_ref[...], sin_k_ref[...])
            for b in range(B_LOC):
                for hh in range(H_LOC):
                    rsl = slice(b * SQ, (b + 1) * SQ)
                    csl = slice(hh * DH, (hh + 1) * DH)
                    s = lax.dot_general(
                        qr[rsl, csl], kr[rsl, csl],
                        (((1,), (1,)), ((), ())),
                        preferred_element_type=jnp.float32)
                    e = jnp.exp(s)
                    recip = 1.0 / jnp.sum(e, axis=-1, keepdims=True)
                    pv = jnp.dot(e.astype(jnp.bfloat16), v[rsl, csl],
                                 preferred_element_type=jnp.float32)
                    ctx[rsl, csl] = (pv * recip).astype(jnp.bfloat16)
            return jnp.dot(ctx[...], wo,
                           preferred_element_type=jnp.float32).astype(jnp.bfloat16)

        acc[0, :, :] = partial_out(xg[0])

        rs_sends = []
        for r in _REMOTE_ORDER:
            recv = pltpu.make_async_remote_copy(
                src_ref=xg.at[r],
                dst_ref=xg.at[r],
                send_sem=ag_send_sems.at[r - 1],
                recv_sem=ag_recv_sems.at[r - 1],
                device_id=(my,),
                device_id_type=pl.DeviceIdType.MESH,
            )
            recv.wait_recv()
            acc[r, :, :] = partial_out(xg[r])
            rdma = pltpu.make_async_remote_copy(
                src_ref=acc.at[r],
                dst_ref=rs_stage.at[r - 1],
                send_sem=rs_send_sems.at[r - 1],
                recv_sem=rs_recv_sems.at[r - 1],
                device_id=((my + N_DEV - r) % N_DEV,),
                device_id_type=pl.DeviceIdType.MESH,
            )
            rdma.start()
            rs_sends.append(rdma)

        for r in _REMOTE_ORDER:
            recv = pltpu.make_async_remote_copy(
                src_ref=rs_stage.at[r - 1],
                dst_ref=rs_stage.at[r - 1],
                send_sem=rs_send_sems.at[r - 1],
                recv_sem=rs_recv_sems.at[r - 1],
                device_id=(my,),
                device_id_type=pl.DeviceIdType.MESH,
            )
            recv.wait_recv()

        out_ref[...] = (acc[0].astype(jnp.float32)
                        + rs_stage[0].astype(jnp.float32)
                        + rs_stage[1].astype(jnp.float32)
                        + rs_stage[2].astype(jnp.float32))

        for rdma in ag_sends + rs_sends:
            rdma.wait_send()

    out = pl.pallas_call(
        body,
        out_shape=jax.ShapeDtypeStruct((ROWS, D), jnp.float32),
        in_specs=[pl.BlockSpec(memory_space=pltpu.VMEM)] * 10,
        out_specs=pl.BlockSpec(memory_space=pltpu.VMEM),
        scratch_shapes=[
            pltpu.VMEM((N_DEV, ROWS, D), jnp.bfloat16),
            pltpu.VMEM((N_DEV, ROWS, D), jnp.bfloat16),
            pltpu.VMEM((N_DEV - 1, ROWS, D), jnp.bfloat16),
            pltpu.VMEM((ROWS, HD_LOC), jnp.bfloat16),
            pltpu.SemaphoreType.DMA((N_DEV - 1,)),
            pltpu.SemaphoreType.DMA((N_DEV - 1,)),
            pltpu.SemaphoreType.DMA((N_DEV - 1,)),
            pltpu.SemaphoreType.DMA((N_DEV - 1,)),
        ],
        compiler_params=pltpu.CompilerParams(collective_id=0),
    )(x2, Wq, Wk, Wv, Wo, cos_q, sin_q, cos_k, sin_k, rmat)
    return out.reshape(B_LOC, SQ, D)
